# baseline (speedup 1.0000x reference)
import functools

import jax
import jax.numpy as jnp
from jax import lax
from jax.experimental import pallas as pl
from jax.experimental.pallas import tpu as pltpu
from jax.experimental.pallas import tpu_sc as plsc

N = 10000
D = 128
E = 320000

NC = 2
NS = 16
L = 16

NW = NC * NS
ET = E // NW
K = 128
NCHUNK = -(-ET // K)
ETP = NCHUNK * K
RB = 80
ACCR = N + RB
NRB = ACCR // RB
NWB = N // RB
RBPT = -(-NRB // NS)
SDN = N + L
NBLK = 10
BR = N // NBLK


EB = E // NBLK


def _matmul_body(x_ref, w_ref, b_ref, ei_ref, ea_ref, o_ref, ob_ref,
                 pk_ref):
    xt = (jnp.dot(x_ref[...], w_ref[...], preferred_element_type=jnp.float32)
          + b_ref[...])
    o_ref[...] = xt
    ob_ref[...] = xt.astype(jnp.bfloat16)
    pk_ref[0:2, :] = ei_ref[...]
    pk_ref[2:3, :] = ea_ref[...]


def _xt_matmul(x, W, b, ei, eab):
    return pl.pallas_call(
        _matmul_body,
        grid=(NBLK,),
        in_specs=[
            pl.BlockSpec((BR, D), lambda i: (i, 0)),
            pl.BlockSpec((D, D), lambda i: (0, 0)),
            pl.BlockSpec((1, D), lambda i: (0, 0)),
            pl.BlockSpec((2, EB), lambda i: (0, i)),
            pl.BlockSpec((1, EB), lambda i: (0, i)),
        ],
        out_specs=[
            pl.BlockSpec((BR, D), lambda i: (i, 0)),
            pl.BlockSpec((BR, D), lambda i: (i, 0)),
            pl.BlockSpec((3, EB), lambda i: (0, i)),
        ],
        out_shape=[
            jax.ShapeDtypeStruct((N, D), jnp.float32),
            jax.ShapeDtypeStruct((N, D), jnp.bfloat16),
            jax.ShapeDtypeStruct((3, E), jnp.int32),
        ],
    )(x, W, b.reshape(1, D), ei, eab)


_sc_mesh = plsc.VectorSubcoreMesh(core_axis_name="c", subcore_axis_name="s")


@functools.partial(
    pl.kernel,
    out_type=(
        jax.ShapeDtypeStruct((NC, N, D), jnp.bfloat16),
        jax.ShapeDtypeStruct((NBLK, NW, BR), jnp.float32),
        jax.ShapeDtypeStruct((NBLK, NW, BR), jnp.float32),
    ),
    mesh=_sc_mesh,
    scratch_types=[
        pltpu.VMEM_SHARED((ACCR, D), jnp.bfloat16),
        pltpu.VMEM((3, K), jnp.int32),
        pltpu.VMEM((3, K), jnp.int32),
        pltpu.VMEM((3, K), jnp.int32),
        pltpu.VMEM((K, D), jnp.bfloat16),
        pltpu.VMEM((K, D), jnp.bfloat16),
        pltpu.VMEM((SDN,), jnp.float32),
        pltpu.VMEM((SDN,), jnp.float32),
        pltpu.SemaphoreType.DMA,
        pltpu.SemaphoreType.DMA,
        pltpu.SemaphoreType.DMA,
        pltpu.SemaphoreType.DMA,
        pltpu.SemaphoreType.DMA,
        pltpu.SemaphoreType.DMA,
        pltpu.SemaphoreType.DMA,
    ],
    compiler_params=pltpu.CompilerParams(use_tc_tiling_on_sc=False,
                                         needs_layout_passes=False),
)
def _sc_scatter(xt_hbm, pk_hbm, agg_hbm, s_hbm, d_hbm,
                acc, pk0, pk1, pk2, rows0, rows1, sv, dv,
                isem0, isem1, isem2, gsem0, gsem1, ssem0, ssem1):
    c = lax.axis_index("c")
    s = lax.axis_index("s")
    wid = c * NS + s
    ebase = wid * ETP

    pks = (pk0, pk1, pk2)
    isems = (isem0, isem1, isem2)
    rows = (rows0, rows1)
    gsems = (gsem0, gsem1)
    ssems = (ssem0, ssem1)

    zeros16 = jnp.zeros((L,), jnp.float32)
    ones16 = jnp.ones((L,), jnp.float32)

    def _load_idx(j, m3):
        pltpu.async_copy(pk_hbm.at[:, pl.ds(ebase + j * K, K)], pks[m3],
                         isems[m3])

    def _wait_idx(m3):
        pltpu.make_async_copy(pk_hbm.at[:, pl.ds(0, K)], pks[m3],
                              isems[m3]).wait()

    def _gather(m3, m2):
        pltpu.async_copy(xt_hbm.at[pks[m3].at[1]], rows[m2], gsems[m2])

    def _wait_gather(m3, m2):
        pltpu.make_async_copy(xt_hbm.at[pks[m3].at[1]], rows[m2],
                              gsems[m2]).wait()

    def _scatter(m3, m2):
        pltpu.async_copy(rows[m2], acc.at[pks[m3].at[0]], ssems[m2],
                         add=True)

    def _wait_scatter(m3, m2):
        pltpu.make_async_copy(rows[m2], acc.at[pks[m3].at[0]],
                              ssems[m2]).wait()

    def _scalars(m3):
        pk = pks[m3]
        for g in range(K // L):
            r16 = pk[0, pl.ds(g * L, L)]
            a16 = plsc.bitcast(pk[2, pl.ds(g * L, L)], jnp.float32)
            plsc.addupdate_scatter(sv, [r16], a16)
            plsc.addupdate_scatter(dv, [r16], ones16)

    _load_idx(0, 0)
    _load_idx(1, 1)

    def _z1(i, carry):
        sv[pl.ds(i * L, L)] = zeros16
        dv[pl.ds(i * L, L)] = zeros16
        return carry
    lax.fori_loop(0, SDN // L, _z1, 0)

    zeros32b = jnp.zeros((2 * L,), jnp.bfloat16)

    def _z2(i, carry):
        r = i // (D // (2 * L))
        j = i % (D // (2 * L))
        rows0[r, pl.ds(j * 2 * L, 2 * L)] = zeros32b
        return carry
    lax.fori_loop(0, K * (D // (2 * L)), _z2, 0)
    for i in range(RBPT):
        j = s + NS * i
        @pl.when(j < NRB)
        def _():
            pltpu.sync_copy(rows0.at[pl.ds(0, RB), :],
                            acc.at[pl.ds(j * RB, RB), :])
    plsc.subcore_barrier()

    def _seg(j, m3, m2, wait_prev=True, next_gather=True, load=True):
        if wait_prev:
            _wait_scatter((m3 + 2) % 3, 1 - m2)
        if next_gather:
            _wait_idx((m3 + 1) % 3)
            _gather((m3 + 1) % 3, 1 - m2)
        _wait_gather(m3, m2)
        _scatter(m3, m2)
        _scalars(m3)
        if load:
            _load_idx(j + 2, (m3 + 2) % 3)

    _wait_idx(0)
    _gather(0, 0)
    _seg(0, 0, 0, wait_prev=False)

    NTAIL = 6
    NSTEADY = NCHUNK - 1 - NTAIL

    def _six(i, carry):
        jb = 1 + 6 * i
        for u in range(6):
            _seg(jb + u, (1 + u) % 3, (1 + u) % 2)
        return carry
    lax.fori_loop(0, NSTEADY // 6, _six, 0)

    for j in range(NCHUNK - NTAIL, NCHUNK):
        _seg(j, j % 3, j % 2,
             next_gather=(j + 1 < NCHUNK), load=(j + 2 < NCHUNK))
    _wait_scatter((NCHUNK - 1) % 3, (NCHUNK - 1) % 2)

    plsc.subcore_barrier()

    for i in range(RBPT):
        j = s + NS * i
        @pl.when(j < NWB)
        def _():
            pltpu.sync_copy(acc.at[pl.ds(j * RB, RB), :],
                            agg_hbm.at[c, pl.ds(j * RB, RB), :])
    for t in range(NBLK):
        pltpu.sync_copy(sv.at[pl.ds(t * BR, BR)], s_hbm.at[t, wid])
        pltpu.sync_copy(dv.at[pl.ds(t * BR, BR)], d_hbm.at[t, wid])


def _epi_body(agg_ref, xt_ref, s_ref, d_ref, we_ref, be_ref, g_ref, bt_ref,
              o_ref):
    agg = (agg_ref[0].astype(jnp.float32) + agg_ref[1].astype(jnp.float32))
    sloc = jnp.sum(s_ref[0], axis=0)
    dloc = jnp.sum(d_ref[0], axis=0)
    out = (agg + xt_ref[...]
           + sloc[:, None] * we_ref[...]
           + dloc[:, None] * be_ref[...])
    a = out * jax.nn.sigmoid(out)
    mean = jnp.mean(a, axis=1, keepdims=True)
    var = jnp.mean((a - mean) ** 2, axis=1, keepdims=True)
    normed = (a - mean) * lax.rsqrt(var + 1e-5)
    o_ref[...] = normed * g_ref[...] + bt_ref[...]


def _epilogue(agg, x_t, s_parts, d_parts, We, be, gamma, beta):
    return pl.pallas_call(
        _epi_body,
        grid=(NBLK,),
        in_specs=[
            pl.BlockSpec((NC, BR, D), lambda i: (0, i, 0)),
            pl.BlockSpec((BR, D), lambda i: (i, 0)),
            pl.BlockSpec((1, NW, BR), lambda i: (i, 0, 0)),
            pl.BlockSpec((1, NW, BR), lambda i: (i, 0, 0)),
            pl.BlockSpec((1, D), lambda i: (0, 0)),
            pl.BlockSpec((1, D), lambda i: (0, 0)),
            pl.BlockSpec((1, D), lambda i: (0, 0)),
            pl.BlockSpec((1, D), lambda i: (0, 0)),
        ],
        out_specs=pl.BlockSpec((BR, D), lambda i: (i, 0)),
        out_shape=jax.ShapeDtypeStruct((N, D), jnp.float32),
    )(agg, x_t, s_parts, d_parts, We.reshape(1, D), be.reshape(1, D),
      gamma.reshape(1, D), beta.reshape(1, D))


def kernel(x, edge_index, edge_attr, W, b, We, be, gamma, beta):
    ei = edge_index.astype(jnp.int32)
    eab = lax.bitcast_convert_type(edge_attr[:, 0], jnp.int32)[None, :]
    x_t, x_tb, pk = _xt_matmul(x, W, b, ei, eab)
    pk3 = pk.reshape(3, NW, ET)
    padv = jnp.array([N, 0, 0], jnp.int32).reshape(3, 1, 1)
    pad = jnp.broadcast_to(padv, (3, NW, ETP - ET))
    pkp = jnp.concatenate([pk3, pad], axis=2).reshape(3, NW * ETP)
    agg, s_parts, d_parts = _sc_scatter(x_tb, pkp)
    return _epilogue(agg, x_t, s_parts, d_parts, We, be, gamma, beta)


# speedup vs baseline: 1.2661x; 1.2661x over previous
import functools

import jax
import jax.numpy as jnp
from jax import lax
from jax.experimental import pallas as pl
from jax.experimental.pallas import tpu as pltpu
from jax.experimental.pallas import tpu_sc as plsc

N = 10000
D = 128
E = 320000

NC = 2
NS = 16
L = 16

NW = NC * NS
ET = E // NW
K = 80
NCHUNK = ET // K
RB = 80
NRB = N // RB
RBPT = -(-NRB // NS)
NBLK = 10
BR = N // NBLK


EB = E // NBLK


def _matmul_body(x_ref, w_ref, b_ref, ei_ref, ea_ref, o_ref, ob_ref,
                 pk_ref):
    xt = (jnp.dot(x_ref[...], w_ref[...], preferred_element_type=jnp.float32)
          + b_ref[...])
    o_ref[...] = xt
    ob_ref[...] = xt.astype(jnp.bfloat16)
    pk_ref[0:2, :] = ei_ref[...]
    pk_ref[2:3, :] = ea_ref[...]


def _xt_matmul(x, W, b, ei, eab):
    return pl.pallas_call(
        _matmul_body,
        grid=(NBLK,),
        in_specs=[
            pl.BlockSpec((BR, D), lambda i: (i, 0)),
            pl.BlockSpec((D, D), lambda i: (0, 0)),
            pl.BlockSpec((1, D), lambda i: (0, 0)),
            pl.BlockSpec((2, EB), lambda i: (0, i)),
            pl.BlockSpec((1, EB), lambda i: (0, i)),
        ],
        out_specs=[
            pl.BlockSpec((BR, D), lambda i: (i, 0)),
            pl.BlockSpec((BR, D), lambda i: (i, 0)),
            pl.BlockSpec((3, EB), lambda i: (0, i)),
        ],
        out_shape=[
            jax.ShapeDtypeStruct((N, D), jnp.float32),
            jax.ShapeDtypeStruct((N, D), jnp.bfloat16),
            jax.ShapeDtypeStruct((3, E), jnp.int32),
        ],
    )(x, W, b.reshape(1, D), ei, eab)


_sc_mesh = plsc.VectorSubcoreMesh(core_axis_name="c", subcore_axis_name="s")


@functools.partial(
    pl.kernel,
    out_type=(
        jax.ShapeDtypeStruct((NC, N, D), jnp.bfloat16),
        jax.ShapeDtypeStruct((NBLK, NW, BR), jnp.float32),
        jax.ShapeDtypeStruct((NBLK, NW, BR), jnp.float32),
    ),
    mesh=_sc_mesh,
    scratch_types=[
        pltpu.VMEM_SHARED((N, D), jnp.bfloat16),
        pltpu.VMEM((3, K), jnp.int32),
        pltpu.VMEM((3, K), jnp.int32),
        pltpu.VMEM((3, K), jnp.int32),
        pltpu.VMEM((K, D), jnp.bfloat16),
        pltpu.VMEM((K, D), jnp.bfloat16),
        pltpu.VMEM((K, D), jnp.bfloat16),
        pltpu.VMEM((N,), jnp.float32),
        pltpu.VMEM((N,), jnp.float32),
        pltpu.SemaphoreType.DMA,
        pltpu.SemaphoreType.DMA,
        pltpu.SemaphoreType.DMA,
        pltpu.SemaphoreType.DMA,
        pltpu.SemaphoreType.DMA,
        pltpu.SemaphoreType.DMA,
        pltpu.SemaphoreType.DMA,
        pltpu.SemaphoreType.DMA,
        pltpu.SemaphoreType.DMA,
    ],
    compiler_params=pltpu.CompilerParams(use_tc_tiling_on_sc=False,
                                         needs_layout_passes=False),
)
def _sc_scatter(xt_hbm, pk_hbm, agg_hbm, s_hbm, d_hbm,
                acc, pk0, pk1, pk2, rows0, rows1, rows2, sv, dv,
                isem0, isem1, isem2, gsem0, gsem1, gsem2,
                ssem0, ssem1, ssem2):
    c = lax.axis_index("c")
    s = lax.axis_index("s")
    wid = c * NS + s
    ebase = wid * ET

    pks = (pk0, pk1, pk2)
    isems = (isem0, isem1, isem2)
    rows = (rows0, rows1, rows2)
    gsems = (gsem0, gsem1, gsem2)
    ssems = (ssem0, ssem1, ssem2)

    zeros16 = jnp.zeros((L,), jnp.float32)
    ones16 = jnp.ones((L,), jnp.float32)

    def _load_idx(j, m3):
        pltpu.async_copy(pk_hbm.at[:, pl.ds(ebase + j * K, K)], pks[m3],
                         isems[m3])

    def _wait_idx(m3):
        pltpu.make_async_copy(pk_hbm.at[:, pl.ds(0, K)], pks[m3],
                              isems[m3]).wait()

    def _gather(m):
        pltpu.async_copy(xt_hbm.at[pks[m].at[1]], rows[m], gsems[m])

    def _wait_gather(m):
        pltpu.make_async_copy(xt_hbm.at[pks[m].at[1]], rows[m],
                              gsems[m]).wait()

    def _scatter(m):
        pltpu.async_copy(rows[m], acc.at[pks[m].at[0]], ssems[m],
                         add=True)

    def _wait_scatter(m):
        pltpu.make_async_copy(rows[m], acc.at[pks[m].at[0]],
                              ssems[m]).wait()

    def _scalars(m3):
        pk = pks[m3]
        for g in range(K // L):
            r16 = pk[0, pl.ds(g * L, L)]
            a16 = plsc.bitcast(pk[2, pl.ds(g * L, L)], jnp.float32)
            plsc.addupdate_scatter(sv, [r16], a16)
            plsc.addupdate_scatter(dv, [r16], ones16)

    _load_idx(0, 0)
    _load_idx(1, 1)

    def _z1(i, carry):
        sv[pl.ds(i * L, L)] = zeros16
        dv[pl.ds(i * L, L)] = zeros16
        return carry
    lax.fori_loop(0, N // L, _z1, 0)

    zeros32b = jnp.zeros((2 * L,), jnp.bfloat16)

    def _z2(i, carry):
        r = i // (D // (2 * L))
        j = i % (D // (2 * L))
        rows0[r, pl.ds(j * 2 * L, 2 * L)] = zeros32b
        return carry
    lax.fori_loop(0, K * (D // (2 * L)), _z2, 0)
    for i in range(RBPT):
        j = s + NS * i
        @pl.when(j < NRB)
        def _():
            pltpu.sync_copy(rows0, acc.at[pl.ds(j * RB, RB), :])
    plsc.subcore_barrier()

    def _seg(j, m, wait_prev=True, next_gather=True, load=True):
        if next_gather:
            _wait_idx((m + 1) % 3)
            _gather((m + 1) % 3)
        _wait_gather(m)
        _scatter(m)
        _scalars(m)
        if wait_prev:
            _wait_scatter((m + 2) % 3)
        if load:
            _load_idx(j + 2, (m + 2) % 3)

    _wait_idx(0)
    _gather(0)
    _seg(0, 0, wait_prev=False, load=False)
    _load_idx(2, 2)
    _seg(1, 1)

    def _three(i, carry):
        jb = 2 + 3 * i
        for u in range(3):
            _seg(jb + u, (2 + u) % 3)
        return carry
    lax.fori_loop(0, (NCHUNK - 5) // 3, _three, 0)

    _seg(122, 122 % 3)
    _seg(123, 123 % 3, load=False)
    _seg(124, 124 % 3, next_gather=False, load=False)
    _wait_scatter(124 % 3)

    plsc.subcore_barrier()

    for i in range(RBPT):
        j = s + NS * i
        @pl.when(j < NRB)
        def _():
            pltpu.sync_copy(acc.at[pl.ds(j * RB, RB), :],
                            agg_hbm.at[c, pl.ds(j * RB, RB), :])
    for t in range(NBLK):
        pltpu.sync_copy(sv.at[pl.ds(t * BR, BR)], s_hbm.at[t, wid])
        pltpu.sync_copy(dv.at[pl.ds(t * BR, BR)], d_hbm.at[t, wid])


def _epi_body(agg_ref, xt_ref, s_ref, d_ref, we_ref, be_ref, g_ref, bt_ref,
              o_ref):
    agg = agg_ref[0].astype(jnp.float32) + agg_ref[1].astype(jnp.float32)
    sloc = jnp.sum(s_ref[0], axis=0)
    dloc = jnp.sum(d_ref[0], axis=0)
    out = (agg + xt_ref[...]
           + sloc[:, None] * we_ref[...]
           + dloc[:, None] * be_ref[...])
    a = out * jax.nn.sigmoid(out)
    mean = jnp.mean(a, axis=1, keepdims=True)
    var = jnp.mean((a - mean) ** 2, axis=1, keepdims=True)
    normed = (a - mean) * lax.rsqrt(var + 1e-5)
    o_ref[...] = normed * g_ref[...] + bt_ref[...]


def _epilogue(agg, x_t, s_parts, d_parts, We, be, gamma, beta):
    return pl.pallas_call(
        _epi_body,
        grid=(NBLK,),
        in_specs=[
            pl.BlockSpec((NC, BR, D), lambda i: (0, i, 0)),
            pl.BlockSpec((BR, D), lambda i: (i, 0)),
            pl.BlockSpec((1, NW, BR), lambda i: (i, 0, 0)),
            pl.BlockSpec((1, NW, BR), lambda i: (i, 0, 0)),
            pl.BlockSpec((1, D), lambda i: (0, 0)),
            pl.BlockSpec((1, D), lambda i: (0, 0)),
            pl.BlockSpec((1, D), lambda i: (0, 0)),
            pl.BlockSpec((1, D), lambda i: (0, 0)),
        ],
        out_specs=pl.BlockSpec((BR, D), lambda i: (i, 0)),
        out_shape=jax.ShapeDtypeStruct((N, D), jnp.float32),
    )(agg, x_t, s_parts, d_parts, We.reshape(1, D), be.reshape(1, D),
      gamma.reshape(1, D), beta.reshape(1, D))


def kernel(x, edge_index, edge_attr, W, b, We, be, gamma, beta):
    ei = edge_index.astype(jnp.int32)
    eab = lax.bitcast_convert_type(edge_attr[:, 0], jnp.int32)[None, :]
    x_t, x_tb, pk = _xt_matmul(x, W, b, ei, eab)
    agg, s_parts, d_parts = _sc_scatter(x_tb, pk)
    return _epilogue(agg, x_t, s_parts, d_parts, We, be, gamma, beta)


# speedup vs baseline: 1.2929x; 1.0211x over previous
import functools

import jax
import jax.numpy as jnp
from jax import lax
from jax.experimental import pallas as pl
from jax.experimental.pallas import tpu as pltpu
from jax.experimental.pallas import tpu_sc as plsc

N = 10000
D = 128
E = 320000

NC = 2
NS = 16
L = 16

NW = NC * NS
ET = E // NW
K = 80
NCHUNK = ET // K
RB = 80
NRB = N // RB
RBPT = -(-NRB // NS)
NBLK = 10
BR = N // NBLK


EB = E // NBLK


def _matmul_body(x_ref, w_ref, b_ref, ei_ref, ea_ref, o_ref, pk_ref):
    o_ref[...] = (
        jnp.dot(x_ref[...], w_ref[...], preferred_element_type=jnp.float32)
        + b_ref[...]
    )
    pk_ref[0:2, :] = ei_ref[...]
    pk_ref[2:3, :] = ea_ref[...]


def _xt_matmul(x, W, b, ei, eab):
    return pl.pallas_call(
        _matmul_body,
        grid=(NBLK,),
        in_specs=[
            pl.BlockSpec((BR, D), lambda i: (i, 0)),
            pl.BlockSpec((D, D), lambda i: (0, 0)),
            pl.BlockSpec((1, D), lambda i: (0, 0)),
            pl.BlockSpec((2, EB), lambda i: (0, i)),
            pl.BlockSpec((1, EB), lambda i: (0, i)),
        ],
        out_specs=[
            pl.BlockSpec((BR, D), lambda i: (i, 0)),
            pl.BlockSpec((3, EB), lambda i: (0, i)),
        ],
        out_shape=[
            jax.ShapeDtypeStruct((N, D), jnp.float32),
            jax.ShapeDtypeStruct((3, E), jnp.int32),
        ],
    )(x, W, b.reshape(1, D), ei, eab)


_sc_mesh = plsc.VectorSubcoreMesh(core_axis_name="c", subcore_axis_name="s")


@functools.partial(
    pl.kernel,
    out_type=(
        jax.ShapeDtypeStruct((NC, N, D), jnp.float32),
        jax.ShapeDtypeStruct((NBLK, NW, BR), jnp.float32),
        jax.ShapeDtypeStruct((NBLK, NW, BR), jnp.float32),
    ),
    mesh=_sc_mesh,
    scratch_types=[
        pltpu.VMEM_SHARED((N, D), jnp.float32),
        pltpu.VMEM((3, K), jnp.int32),
        pltpu.VMEM((3, K), jnp.int32),
        pltpu.VMEM((3, K), jnp.int32),
        pltpu.VMEM((K, D), jnp.float32),
        pltpu.VMEM((K, D), jnp.float32),
        pltpu.VMEM((N,), jnp.float32),
        pltpu.VMEM((N,), jnp.float32),
        pltpu.SemaphoreType.DMA,
        pltpu.SemaphoreType.DMA,
        pltpu.SemaphoreType.DMA,
        pltpu.SemaphoreType.DMA,
        pltpu.SemaphoreType.DMA,
        pltpu.SemaphoreType.DMA,
        pltpu.SemaphoreType.DMA,
    ],
    compiler_params=pltpu.CompilerParams(use_tc_tiling_on_sc=False,
                                         needs_layout_passes=False),
)
def _sc_scatter(xt_hbm, pk_hbm, agg_hbm, s_hbm, d_hbm,
                acc, pk0, pk1, pk2, rows0, rows1, sv, dv,
                isem0, isem1, isem2, gsem0, gsem1, ssem0, ssem1):
    c = lax.axis_index("c")
    s = lax.axis_index("s")
    wid = c * NS + s
    ebase = wid * ET

    pks = (pk0, pk1, pk2)
    isems = (isem0, isem1, isem2)
    rows = (rows0, rows1)
    gsems = (gsem0, gsem1)
    ssems = (ssem0, ssem1)

    zeros16 = jnp.zeros((L,), jnp.float32)
    ones16 = jnp.ones((L,), jnp.float32)

    def _load_idx(j, m3):
        pltpu.async_copy(pk_hbm.at[:, pl.ds(ebase + j * K, K)], pks[m3],
                         isems[m3])

    def _wait_idx(m3):
        pltpu.make_async_copy(pk_hbm.at[:, pl.ds(0, K)], pks[m3],
                              isems[m3]).wait()

    def _gather(m3, m2):
        pltpu.async_copy(xt_hbm.at[pks[m3].at[1]], rows[m2], gsems[m2])

    def _wait_gather(m3, m2):
        pltpu.make_async_copy(xt_hbm.at[pks[m3].at[1]], rows[m2],
                              gsems[m2]).wait()

    def _scatter(m3, m2):
        pltpu.async_copy(rows[m2], acc.at[pks[m3].at[0]], ssems[m2],
                         add=True)

    def _wait_scatter(m3, m2):
        pltpu.make_async_copy(rows[m2], acc.at[pks[m3].at[0]],
                              ssems[m2]).wait()

    def _scalars(m3):
        pk = pks[m3]
        for g in range(K // L):
            r16 = pk[0, pl.ds(g * L, L)]
            a16 = plsc.bitcast(pk[2, pl.ds(g * L, L)], jnp.float32)
            plsc.addupdate_scatter(sv, [r16], a16)
            plsc.addupdate_scatter(dv, [r16], ones16)

    _load_idx(0, 0)
    _load_idx(1, 1)

    def _z1(i, carry):
        sv[pl.ds(i * L, L)] = zeros16
        dv[pl.ds(i * L, L)] = zeros16
        return carry
    lax.fori_loop(0, N // L, _z1, 0)

    def _z2(i, carry):
        r = i // (D // L)
        j = i % (D // L)
        rows0[r, pl.ds(j * L, L)] = zeros16
        return carry
    lax.fori_loop(0, K * (D // L), _z2, 0)
    for i in range(RBPT):
        j = s + NS * i
        @pl.when(j < NRB)
        def _():
            pltpu.sync_copy(rows0, acc.at[pl.ds(j * RB, RB), :])
    plsc.subcore_barrier()

    def _seg(j, m3, m2, wait_prev=True, next_gather=True, load=True):
        if wait_prev:
            _wait_scatter((m3 + 2) % 3, 1 - m2)
        if next_gather:
            _wait_idx((m3 + 1) % 3)
            _gather((m3 + 1) % 3, 1 - m2)
        _wait_gather(m3, m2)
        _scatter(m3, m2)
        _scalars(m3)
        if load:
            _load_idx(j + 2, (m3 + 2) % 3)

    _wait_idx(0)
    _gather(0, 0)
    _seg(0, 0, 0, wait_prev=False)

    def _six(i, carry):
        jb = 1 + 6 * i
        for u in range(6):
            _seg(jb + u, (1 + u) % 3, (1 + u) % 2)
        return carry
    lax.fori_loop(0, (NCHUNK - 5) // 6, _six, 0)

    _seg(121, (121 % 3), (121 % 2))
    _seg(122, (122 % 3), (122 % 2))
    _seg(123, (123 % 3), (123 % 2), load=False)
    _seg(124, (124 % 3), (124 % 2), next_gather=False, load=False)
    _wait_scatter(124 % 3, 124 % 2)

    plsc.subcore_barrier()

    for i in range(RBPT):
        j = s + NS * i
        @pl.when(j < NRB)
        def _():
            pltpu.sync_copy(acc.at[pl.ds(j * RB, RB), :],
                            agg_hbm.at[c, pl.ds(j * RB, RB), :])
    for t in range(NBLK):
        pltpu.sync_copy(sv.at[pl.ds(t * BR, BR)], s_hbm.at[t, wid])
        pltpu.sync_copy(dv.at[pl.ds(t * BR, BR)], d_hbm.at[t, wid])


def _epi_body(agg_ref, xt_ref, s_ref, d_ref, we_ref, be_ref, g_ref, bt_ref,
              o_ref):
    agg = agg_ref[0] + agg_ref[1]
    sloc = jnp.sum(s_ref[0], axis=0)
    dloc = jnp.sum(d_ref[0], axis=0)
    out = (agg + xt_ref[...]
           + sloc[:, None] * we_ref[...]
           + dloc[:, None] * be_ref[...])
    a = out * jax.nn.sigmoid(out)
    mean = jnp.mean(a, axis=1, keepdims=True)
    var = jnp.mean((a - mean) ** 2, axis=1, keepdims=True)
    normed = (a - mean) * lax.rsqrt(var + 1e-5)
    o_ref[...] = normed * g_ref[...] + bt_ref[...]


def _epilogue(agg, x_t, s_parts, d_parts, We, be, gamma, beta):
    return pl.pallas_call(
        _epi_body,
        grid=(NBLK,),
        in_specs=[
            pl.BlockSpec((NC, BR, D), lambda i: (0, i, 0)),
            pl.BlockSpec((BR, D), lambda i: (i, 0)),
            pl.BlockSpec((1, NW, BR), lambda i: (i, 0, 0)),
            pl.BlockSpec((1, NW, BR), lambda i: (i, 0, 0)),
            pl.BlockSpec((1, D), lambda i: (0, 0)),
            pl.BlockSpec((1, D), lambda i: (0, 0)),
            pl.BlockSpec((1, D), lambda i: (0, 0)),
            pl.BlockSpec((1, D), lambda i: (0, 0)),
        ],
        out_specs=pl.BlockSpec((BR, D), lambda i: (i, 0)),
        out_shape=jax.ShapeDtypeStruct((N, D), jnp.float32),
    )(agg, x_t, s_parts, d_parts, We.reshape(1, D), be.reshape(1, D),
      gamma.reshape(1, D), beta.reshape(1, D))


def kernel(x, edge_index, edge_attr, W, b, We, be, gamma, beta):
    ei = edge_index.astype(jnp.int32)
    eab = lax.bitcast_convert_type(edge_attr[:, 0], jnp.int32)[None, :]
    x_t, pk = _xt_matmul(x, W, b, ei, eab)
    agg, s_parts, d_parts = _sc_scatter(x_t, pk)
    return _epilogue(agg, x_t, s_parts, d_parts, We, be, gamma, beta)
